# baseline (speedup 1.0000x reference)
import functools

import jax
import jax.numpy as jnp
from jax import lax
from jax.experimental import pallas as pl
from jax.experimental.pallas import tpu as pltpu
from jax.experimental.pallas import tpu_sc as plsc

N = 4096
E = 65536
DX = 512
H = 1024
CE = 5
CY = 10

NC = 2
NS = 16
NW = NC * NS


def _proj_body(x_ref, w1_ref, w2_ref, o1_ref, o2_ref):
    x = x_ref[...]
    o1_ref[...] = jnp.dot(
        x, w1_ref[...], preferred_element_type=jnp.float32
    ).astype(jnp.bfloat16)
    o2_ref[...] = jnp.dot(
        x, w2_ref[...], preferred_element_type=jnp.float32
    ).astype(jnp.bfloat16)


def _project(X, W1, W2):
    BM = 512
    return pl.pallas_call(
        _proj_body,
        grid=(N // BM,),
        in_specs=[
            pl.BlockSpec((BM, DX), lambda i: (i, 0)),
            pl.BlockSpec((DX, H), lambda i: (0, 0)),
            pl.BlockSpec((DX, H), lambda i: (0, 0)),
        ],
        out_specs=[
            pl.BlockSpec((BM, H), lambda i: (i, 0)),
            pl.BlockSpec((BM, H), lambda i: (i, 0)),
        ],
        out_shape=[jax.ShapeDtypeStruct((N, H), jnp.bfloat16)] * 2,
    )(X, W1, W2)


_CH = 32
_HW = H // 2


def _sc_gather(T1, T2, src, dst):
    mesh = plsc.VectorSubcoreMesh(core_axis_name="c", subcore_axis_name="s")
    bpw = E // NW
    nch = bpw // _CH

    @functools.partial(
        pl.kernel,
        out_type=[jax.ShapeDtypeStruct((E, _HW), jnp.int32)] * 2,
        mesh=mesh,
        scratch_types=[
            pltpu.VMEM((bpw,), jnp.int32),
            pltpu.VMEM((bpw,), jnp.int32),
            pltpu.VMEM((_CH, _HW), jnp.int32),
            pltpu.VMEM((_CH, _HW), jnp.int32),
            pltpu.VMEM((_CH, _HW), jnp.int32),
            pltpu.VMEM((_CH, _HW), jnp.int32),
            pltpu.SemaphoreType.DMA,
            pltpu.SemaphoreType.DMA,
            pltpu.SemaphoreType.DMA,
            pltpu.SemaphoreType.DMA,
        ],
    )
    def gather_kernel(t1, t2, s_h, d_h, g1, g2, s_all, d_all,
                      r1a, r2a, r1b, r2b, m1a, m2a, m1b, m2b):
        wid = lax.axis_index("s") * NC + lax.axis_index("c")
        base = wid * bpw
        pltpu.sync_copy(s_h.at[pl.ds(base, bpw)], s_all)
        pltpu.sync_copy(d_h.at[pl.ds(base, bpw)], d_all)

        def issue(c, r1, r2, m1, m2):
            pltpu.async_copy(t1.at[s_all.at[pl.ds(c * _CH, _CH)]], r1, m1)
            pltpu.async_copy(t2.at[d_all.at[pl.ds(c * _CH, _CH)]], r2, m2)

        def wait(c, r1, r2, m1, m2):
            i1 = s_all.at[pl.ds(c * _CH, _CH)]
            i2 = d_all.at[pl.ds(c * _CH, _CH)]
            pltpu.make_async_copy(t1.at[i1], r1, m1).wait()
            pltpu.make_async_copy(t2.at[i2], r2, m2).wait()

        def wout(c, r1, r2):
            off = base + c * _CH
            pltpu.sync_copy(r1, g1.at[pl.ds(off, _CH)])
            pltpu.sync_copy(r2, g2.at[pl.ds(off, _CH)])

        issue(0, r1a, r2a, m1a, m2a)

        def body(c2, carry):
            cA = 2 * c2
            cB = cA + 1
            issue(cB, r1b, r2b, m1b, m2b)
            wait(cA, r1a, r2a, m1a, m2a)
            wout(cA, r1a, r2a)

            @pl.when(cA + 2 < nch)
            def _():
                issue(cA + 2, r1a, r2a, m1a, m2a)

            wait(cB, r1b, r2b, m1b, m2b)
            wout(cB, r1b, r2b)
            return carry

        lax.fori_loop(0, nch // 2, body, 0)

    return gather_kernel(T1, T2, src, dst)


_TE = 2048


def _edge_head(G1, G2, At, tf, W5, W6, be, Wo, bo, gum, eoh):
    grid = (E // _TE,)

    def body(g1_ref, g2_ref, at_ref, t_ref, w5_ref, w6_ref, be_ref, wo_ref,
             bo_ref, gum_ref, eoh_ref, b_ref, le_ref):
        i = pl.program_id(0)
        sm = jnp.dot(at_ref[...], w5_ref[...],
                     preferred_element_type=jnp.float32)
        sm = sm + t_ref[...] * w6_ref[...]
        g = g1_ref[...].astype(jnp.float32) + g2_ref[...].astype(jnp.float32)
        h = jnp.maximum(g + sm + be_ref[...], 0.0)
        logit = jnp.dot(h, wo_ref[...],
                        preferred_element_type=jnp.float32) + bo_ref[...]
        col = lax.broadcasted_iota(jnp.int32, (_TE, CE), 1)
        z = logit + gum_ref[...]
        zmax = jnp.max(z, axis=1, keepdims=True)
        samp = jnp.min(jnp.where(z >= zmax, col, CE), axis=1)
        b_ref[...] = (samp != 0).astype(jnp.float32)[None, None, :]
        eoh = eoh_ref[...]
        emax = jnp.max(eoh, axis=1, keepdims=True)
        te_idx = jnp.min(jnp.where(eoh >= emax, col, CE), axis=1)
        lmax = jnp.max(logit, axis=1, keepdims=True)
        lse = jnp.log(jnp.sum(jnp.exp(logit - lmax), axis=1)) + lmax[:, 0]
        lp_t = jnp.sum(jnp.where(col == te_idx[:, None], logit, 0.0),
                       axis=1) - lse
        part = -jnp.sum(lp_t) * (1.0 / E)

        @pl.when(i == 0)
        def _():
            le_ref[...] = jnp.zeros_like(le_ref)

        le_ref[...] += part[None, None]

    return pl.pallas_call(
        body,
        grid=grid,
        in_specs=[
            pl.BlockSpec((_TE, H), lambda i: (i, 0)),
            pl.BlockSpec((_TE, H), lambda i: (i, 0)),
            pl.BlockSpec((_TE, CE), lambda i: (i, 0)),
            pl.BlockSpec((_TE, 1), lambda i: (i, 0)),
            pl.BlockSpec((CE, H), lambda i: (0, 0)),
            pl.BlockSpec((1, H), lambda i: (0, 0)),
            pl.BlockSpec((1, H), lambda i: (0, 0)),
            pl.BlockSpec((H, CE), lambda i: (0, 0)),
            pl.BlockSpec((1, CE), lambda i: (0, 0)),
            pl.BlockSpec((_TE, CE), lambda i: (i, 0)),
            pl.BlockSpec((_TE, CE), lambda i: (i, 0)),
        ],
        out_specs=[
            pl.BlockSpec((1, 1, _TE), lambda i: (i, 0, 0)),
            pl.BlockSpec((1, 1), lambda i: (0, 0)),
        ],
        out_shape=[
            jax.ShapeDtypeStruct((E // _TE, 1, _TE), jnp.float32),
            jax.ShapeDtypeStruct((1, 1), jnp.float32),
        ],
    )(G1, G2, At, tf, W5, W6, be, Wo, bo, gum, eoh)


_ZCH = 16384
_SCB = 128
NPAD = N + 32


def _sc_scatter(src, dst, bvals):
    mesh = plsc.VectorSubcoreMesh(core_axis_name="c", subcore_axis_name="s")
    epw = E // NS
    n_sc = epw // _SCB
    half_words = (N // 2) * N

    @functools.partial(
        pl.kernel,
        out_type=jax.ShapeDtypeStruct((NPAD * N,), jnp.float32),
        mesh=mesh,
        scratch_types=[
            pltpu.VMEM((_ZCH,), jnp.float32),
            pltpu.VMEM((epw,), jnp.int32),
            pltpu.VMEM((epw,), jnp.int32),
            pltpu.VMEM((epw,), jnp.float32),
            pltpu.VMEM((n_sc, _SCB), jnp.int32),
            pltpu.VMEM((n_sc, _SCB), jnp.int32),
            pltpu.SemaphoreType.DMA,
            pltpu.SemaphoreType.DMA,
        ],
    )
    def scatter_kernel(s_h, d_h, b_h, a_h, z_v, s_v, d_v, v_v, i1_v, i2_v,
                       sem1, sem2):
        cid = lax.axis_index("c")
        sid = lax.axis_index("s")

        def zb(i, carry):
            z_v[pl.ds(i * 16, 16)] = jnp.zeros((16,), jnp.float32)
            return carry

        lax.fori_loop(0, _ZCH // 16, zb, 0)
        words = half_words // NS
        zbase = cid * half_words + sid * words

        def zc(i, carry):
            pltpu.sync_copy(z_v, a_h.at[pl.ds(zbase + i * _ZCH, _ZCH)])
            return carry

        lax.fori_loop(0, words // _ZCH, zc, 0)
        tzbase = N * N + cid * (16 * N) + sid * N
        pltpu.sync_copy(z_v.at[pl.ds(0, N)], a_h.at[pl.ds(tzbase, N)])

        plsc.subcore_barrier()

        ebase = sid * epw
        pltpu.sync_copy(s_h.at[pl.ds(ebase, epw)], s_v)
        pltpu.sync_copy(d_h.at[pl.ds(ebase, epw)], d_v)
        pltpu.sync_copy(b_h.at[pl.ds(ebase, epw)], v_v)

        lo = cid * (N // 2)
        hi = lo + (N // 2)
        trash = N * N + cid * (16 * N)

        def ixrow(j, carry):
            def ix(i, c2):
                sv = s_v[pl.ds(j * _SCB + i * 16, 16)]
                dv = d_v[pl.ds(j * _SCB + i * 16, 16)]
                own1 = jnp.logical_and(dv >= lo, dv < hi)
                own2 = jnp.logical_and(sv >= lo, sv < hi)
                i1_v[j, pl.ds(i * 16, 16)] = jnp.where(
                    own1, dv * N + sv, trash)
                i2_v[j, pl.ds(i * 16, 16)] = jnp.where(
                    own2, sv * N + dv, trash)
                return c2

            lax.fori_loop(0, _SCB // 16, ix, 0)
            return carry

        lax.fori_loop(0, n_sc, ixrow, 0)

        def sc(j, carry):
            vseg = v_v.at[pl.ds(j * _SCB, _SCB)]
            pltpu.async_copy(vseg, a_h.at[i1_v.at[j]], sem1)
            pltpu.async_copy(vseg, a_h.at[i2_v.at[j]], sem2)
            return carry

        lax.fori_loop(0, n_sc, sc, 0)

        def drain(j, carry):
            vseg = v_v.at[pl.ds(j * _SCB, _SCB)]
            pltpu.make_async_copy(vseg, a_h.at[i1_v.at[j]], sem1).wait()
            pltpu.make_async_copy(vseg, a_h.at[i2_v.at[j]], sem2).wait()
            return carry

        lax.fori_loop(0, n_sc, drain, 0)

    return scatter_kernel(src, dst, bvals)


_BM5 = 256


def _classifier(A, X, Wc1, Wc2, Y3):
    def body(a_ref, x_ref, w1_ref, w2_ref, y_ref, ly_ref):
        i = pl.program_id(0)
        a = a_ref[...]
        row = lax.broadcasted_iota(jnp.int32, (_BM5, N), 0) + i * _BM5
        coln = lax.broadcasted_iota(jnp.int32, (_BM5, N), 1)
        a = jnp.maximum(a, (row == coln).astype(jnp.float32))
        agg = jnp.dot(a.astype(jnp.bfloat16),
                      x_ref[...].astype(jnp.bfloat16),
                      preferred_element_type=jnp.float32)
        hy = jnp.maximum(
            jnp.dot(agg.astype(jnp.bfloat16),
                    w1_ref[...].astype(jnp.bfloat16),
                    preferred_element_type=jnp.float32),
            0.0)
        ly = jnp.dot(hy.astype(jnp.bfloat16),
                     w2_ref[...].astype(jnp.bfloat16),
                     preferred_element_type=jnp.float32)
        yb = y_ref[0, 0, :]
        lmax = jnp.max(ly, axis=1, keepdims=True)
        lse = jnp.log(jnp.sum(jnp.exp(ly - lmax), axis=1)) + lmax[:, 0]
        c10 = lax.broadcasted_iota(jnp.int32, (_BM5, CY), 1)
        lp_t = jnp.sum(jnp.where(c10 == yb[:, None], ly, 0.0), axis=1) - lse
        part = -jnp.sum(lp_t) * (1.0 / N)

        @pl.when(i == 0)
        def _():
            ly_ref[...] = jnp.zeros_like(ly_ref)

        ly_ref[...] += part[None, None]

    return pl.pallas_call(
        body,
        grid=(N // _BM5,),
        in_specs=[
            pl.BlockSpec((_BM5, N), lambda i: (i, 0)),
            pl.BlockSpec((N, DX), lambda i: (0, 0)),
            pl.BlockSpec((DX, H), lambda i: (0, 0)),
            pl.BlockSpec((H, CY), lambda i: (0, 0)),
            pl.BlockSpec((1, 1, _BM5), lambda i: (i, 0, 0)),
        ],
        out_specs=pl.BlockSpec((1, 1), lambda i: (0, 0)),
        out_shape=jax.ShapeDtypeStruct((1, 1), jnp.float32),
    )(A, X, Wc1, Wc2, Y3)


def kernel(X_one_hot_2d, A_t, Y, t_float_E, batch_src, batch_dst,
           batch_E_one_hot, W_e, b_e, W_o, b_o, Wc1, Wc2):
    src = batch_src.astype(jnp.int32)
    dst = batch_dst.astype(jnp.int32)
    W1 = W_e[:DX]
    W2 = W_e[DX:2 * DX]
    W5 = W_e[2 * DX:2 * DX + CE]
    W6 = W_e[2 * DX + CE:].reshape(1, H)
    be = b_e.reshape(1, H)
    bo = b_o.reshape(1, CE)
    gum = jax.random.gumbel(jax.random.key(42), (E, CE), jnp.float32)

    XW1, XW2 = _project(X_one_hot_2d, W1, W2)
    T1 = lax.bitcast_convert_type(XW1.reshape(N, _HW, 2), jnp.int32)
    T2 = lax.bitcast_convert_type(XW2.reshape(N, _HW, 2), jnp.int32)
    G1_i, G2_i = _sc_gather(T1, T2, src, dst)
    G1 = lax.bitcast_convert_type(G1_i, jnp.bfloat16).reshape(E, H)
    G2 = lax.bitcast_convert_type(G2_i, jnp.bfloat16).reshape(E, H)
    bflag3, loss_e = _edge_head(G1, G2, A_t, t_float_E, W5, W6, be, W_o, bo,
                                gum, batch_E_one_hot)
    bflag = bflag3.reshape(E)
    A_flat = _sc_scatter(src, dst, bflag)
    A = A_flat.reshape(NPAD, N)
    Y3 = Y.astype(jnp.int32).reshape(N // _BM5, 1, _BM5)
    loss_y = _classifier(A, X_one_hot_2d, Wc1, Wc2, Y3)
    return loss_e[0, 0], loss_y[0, 0]


# speedup vs baseline: 12.0154x; 12.0154x over previous
import functools

import jax
import jax.numpy as jnp
from jax import lax
from jax.experimental import pallas as pl
from jax.experimental.pallas import tpu as pltpu
from jax.experimental.pallas import tpu_sc as plsc

N = 4096
E = 65536
DX = 512
H = 1024
CE = 5
CY = 10

NC = 2
NS = 16
NW = NC * NS


def _pack_bf16_pair(o):
    u = lax.bitcast_convert_type(o, jnp.uint32)
    b = (u + jnp.uint32(0x7FFF) + ((u >> 16) & jnp.uint32(1))) >> 16
    lo = b[:, :H // 2]
    hi = b[:, H // 2:]
    return lax.bitcast_convert_type(lo | (hi << 16), jnp.int32)


def _unpack_bf16_pair(gi):
    g = lax.bitcast_convert_type(gi, jnp.uint32)
    lo = lax.bitcast_convert_type(g << 16, jnp.float32)
    hi = lax.bitcast_convert_type(g & jnp.uint32(0xFFFF0000), jnp.float32)
    return jnp.concatenate([lo, hi], axis=1)


def _proj_body(x_ref, w1_ref, w2_ref, o1_ref, o2_ref):
    x = x_ref[...]
    o1_ref[...] = _pack_bf16_pair(
        jnp.dot(x, w1_ref[...], preferred_element_type=jnp.float32))
    o2_ref[...] = _pack_bf16_pair(
        jnp.dot(x, w2_ref[...], preferred_element_type=jnp.float32))


def _project(X, W1, W2):
    BM = 512
    return pl.pallas_call(
        _proj_body,
        grid=(N // BM,),
        in_specs=[
            pl.BlockSpec((BM, DX), lambda i: (i, 0)),
            pl.BlockSpec((DX, H), lambda i: (0, 0)),
            pl.BlockSpec((DX, H), lambda i: (0, 0)),
        ],
        out_specs=[
            pl.BlockSpec((BM, H // 2), lambda i: (i, 0)),
            pl.BlockSpec((BM, H // 2), lambda i: (i, 0)),
        ],
        out_shape=[jax.ShapeDtypeStruct((N, H // 2), jnp.int32)] * 2,
    )(X, W1, W2)


_CH = 32
_HW = H // 2


def _sc_gather(T1, T2, src, dst):
    mesh = plsc.VectorSubcoreMesh(core_axis_name="c", subcore_axis_name="s")
    bpw = E // NW
    nch = bpw // _CH

    @functools.partial(
        pl.kernel,
        out_type=[jax.ShapeDtypeStruct((E, _HW), jnp.int32)] * 2,
        mesh=mesh,
        scratch_types=[
            pltpu.VMEM((bpw,), jnp.int32),
            pltpu.VMEM((bpw,), jnp.int32),
            pltpu.VMEM((_CH, _HW), jnp.int32),
            pltpu.VMEM((_CH, _HW), jnp.int32),
            pltpu.VMEM((_CH, _HW), jnp.int32),
            pltpu.VMEM((_CH, _HW), jnp.int32),
            pltpu.SemaphoreType.DMA,
            pltpu.SemaphoreType.DMA,
            pltpu.SemaphoreType.DMA,
            pltpu.SemaphoreType.DMA,
        ],
    )
    def gather_kernel(t1, t2, s_h, d_h, g1, g2, s_all, d_all,
                      r1a, r2a, r1b, r2b, m1a, m2a, m1b, m2b):
        wid = lax.axis_index("s") * NC + lax.axis_index("c")
        base = wid * bpw
        pltpu.sync_copy(s_h.at[pl.ds(base, bpw)], s_all)
        pltpu.sync_copy(d_h.at[pl.ds(base, bpw)], d_all)

        def issue(c, r1, r2, m1, m2):
            pltpu.async_copy(t1.at[s_all.at[pl.ds(c * _CH, _CH)]], r1, m1)
            pltpu.async_copy(t2.at[d_all.at[pl.ds(c * _CH, _CH)]], r2, m2)

        def wait(c, r1, r2, m1, m2):
            i1 = s_all.at[pl.ds(c * _CH, _CH)]
            i2 = d_all.at[pl.ds(c * _CH, _CH)]
            pltpu.make_async_copy(t1.at[i1], r1, m1).wait()
            pltpu.make_async_copy(t2.at[i2], r2, m2).wait()

        def wout(c, r1, r2):
            off = base + c * _CH
            pltpu.sync_copy(r1, g1.at[pl.ds(off, _CH)])
            pltpu.sync_copy(r2, g2.at[pl.ds(off, _CH)])

        issue(0, r1a, r2a, m1a, m2a)

        def body(c2, carry):
            cA = 2 * c2
            cB = cA + 1
            issue(cB, r1b, r2b, m1b, m2b)
            wait(cA, r1a, r2a, m1a, m2a)
            wout(cA, r1a, r2a)

            @pl.when(cA + 2 < nch)
            def _():
                issue(cA + 2, r1a, r2a, m1a, m2a)

            wait(cB, r1b, r2b, m1b, m2b)
            wout(cB, r1b, r2b)
            return carry

        lax.fori_loop(0, nch // 2, body, 0)

    return gather_kernel(T1, T2, src, dst)


_TE = 2048


def _edge_head(G1, G2, At, tf, W5, W6, be, Wo, bo, gum, eoh):
    grid = (E // _TE,)

    def body(g1_ref, g2_ref, at_ref, t_ref, w5_ref, w6_ref, be_ref, wo_ref,
             bo_ref, gum_ref, eoh_ref, b_ref, le_ref):
        i = pl.program_id(0)
        sm = jnp.dot(at_ref[...], w5_ref[...],
                     preferred_element_type=jnp.float32)
        sm = sm + t_ref[...] * w6_ref[...]
        g = _unpack_bf16_pair(g1_ref[...]) + _unpack_bf16_pair(g2_ref[...])
        h = jnp.maximum(g + sm + be_ref[...], 0.0)
        logit = jnp.dot(h, wo_ref[...],
                        preferred_element_type=jnp.float32) + bo_ref[...]
        col = lax.broadcasted_iota(jnp.int32, (_TE, CE), 1)
        z = logit + gum_ref[...]
        zmax = jnp.max(z, axis=1, keepdims=True)
        samp = jnp.min(jnp.where(z >= zmax, col, CE), axis=1)
        b_ref[...] = (samp != 0).astype(jnp.float32)[None, None, :]
        eoh = eoh_ref[...]
        emax = jnp.max(eoh, axis=1, keepdims=True)
        te_idx = jnp.min(jnp.where(eoh >= emax, col, CE), axis=1)
        lmax = jnp.max(logit, axis=1, keepdims=True)
        lse = jnp.log(jnp.sum(jnp.exp(logit - lmax), axis=1)) + lmax[:, 0]
        lp_t = jnp.sum(jnp.where(col == te_idx[:, None], logit, 0.0),
                       axis=1) - lse
        part = -jnp.sum(lp_t) * (1.0 / E)

        @pl.when(i == 0)
        def _():
            le_ref[...] = jnp.zeros_like(le_ref)

        le_ref[...] += part[None, None]

    return pl.pallas_call(
        body,
        grid=grid,
        in_specs=[
            pl.BlockSpec((_TE, _HW), lambda i: (i, 0)),
            pl.BlockSpec((_TE, _HW), lambda i: (i, 0)),
            pl.BlockSpec((_TE, CE), lambda i: (i, 0)),
            pl.BlockSpec((_TE, 1), lambda i: (i, 0)),
            pl.BlockSpec((CE, H), lambda i: (0, 0)),
            pl.BlockSpec((1, H), lambda i: (0, 0)),
            pl.BlockSpec((1, H), lambda i: (0, 0)),
            pl.BlockSpec((H, CE), lambda i: (0, 0)),
            pl.BlockSpec((1, CE), lambda i: (0, 0)),
            pl.BlockSpec((_TE, CE), lambda i: (i, 0)),
            pl.BlockSpec((_TE, CE), lambda i: (i, 0)),
        ],
        out_specs=[
            pl.BlockSpec((1, 1, _TE), lambda i: (i, 0, 0)),
            pl.BlockSpec((1, 1), lambda i: (0, 0)),
        ],
        out_shape=[
            jax.ShapeDtypeStruct((E // _TE, 1, _TE), jnp.float32),
            jax.ShapeDtypeStruct((1, 1), jnp.float32),
        ],
    )(G1, G2, At, tf, W5, W6, be, Wo, bo, gum, eoh)


_ZCH = 16384
_SCB = 128
NPAD = N + 32


def _sc_scatter(src, dst, bvals):
    mesh = plsc.VectorSubcoreMesh(core_axis_name="c", subcore_axis_name="s")
    epw = E // NS
    n_sc = epw // _SCB
    half_words = (N // 2) * N

    @functools.partial(
        pl.kernel,
        out_type=jax.ShapeDtypeStruct((NPAD * N,), jnp.float32),
        mesh=mesh,
        scratch_types=[
            pltpu.VMEM((_ZCH,), jnp.float32),
            pltpu.VMEM((epw,), jnp.int32),
            pltpu.VMEM((epw,), jnp.int32),
            pltpu.VMEM((epw,), jnp.float32),
            pltpu.VMEM((n_sc, _SCB), jnp.int32),
            pltpu.VMEM((n_sc, _SCB), jnp.int32),
            pltpu.SemaphoreType.DMA,
            pltpu.SemaphoreType.DMA,
        ],
    )
    def scatter_kernel(s_h, d_h, b_h, a_h, z_v, s_v, d_v, v_v, i1_v, i2_v,
                       sem1, sem2):
        cid = lax.axis_index("c")
        sid = lax.axis_index("s")

        def zb(i, carry):
            z_v[pl.ds(i * 16, 16)] = jnp.zeros((16,), jnp.float32)
            return carry

        lax.fori_loop(0, _ZCH // 16, zb, 0)
        words = half_words // NS
        zbase = cid * half_words + sid * words

        def zc(i, carry):
            pltpu.sync_copy(z_v, a_h.at[pl.ds(zbase + i * _ZCH, _ZCH)])
            return carry

        lax.fori_loop(0, words // _ZCH, zc, 0)
        tzbase = N * N + cid * (16 * N) + sid * N
        pltpu.sync_copy(z_v.at[pl.ds(0, N)], a_h.at[pl.ds(tzbase, N)])

        plsc.subcore_barrier()

        ebase = sid * epw
        pltpu.sync_copy(s_h.at[pl.ds(ebase, epw)], s_v)
        pltpu.sync_copy(d_h.at[pl.ds(ebase, epw)], d_v)
        pltpu.sync_copy(b_h.at[pl.ds(ebase, epw)], v_v)

        lo = cid * (N // 2)
        hi = lo + (N // 2)
        trash = N * N + cid * (16 * N)
        lane = lax.iota(jnp.int32, 16)

        def ixrow(j, carry):
            def ix(i, c2):
                sv = s_v[pl.ds(j * _SCB + i * 16, 16)]
                dv = d_v[pl.ds(j * _SCB + i * 16, 16)]
                own1 = jnp.logical_and(dv >= lo, dv < hi)
                own2 = jnp.logical_and(sv >= lo, sv < hi)
                tr = trash + jnp.bitwise_and(
                    ebase + j * _SCB + i * 16 + lane, 16 * N - 1)
                i1_v[j, pl.ds(i * 16, 16)] = jnp.where(
                    own1, dv * N + sv, tr)
                i2_v[j, pl.ds(i * 16, 16)] = jnp.where(
                    own2, sv * N + dv, tr)
                return c2

            lax.fori_loop(0, _SCB // 16, ix, 0)
            return carry

        lax.fori_loop(0, n_sc, ixrow, 0)

        def sc(j, carry):
            vseg = v_v.at[pl.ds(j * _SCB, _SCB)]
            pltpu.async_copy(vseg, a_h.at[i1_v.at[j]], sem1)
            pltpu.async_copy(vseg, a_h.at[i2_v.at[j]], sem2)
            return carry

        lax.fori_loop(0, n_sc, sc, 0)

        def drain(j, carry):
            vseg = v_v.at[pl.ds(j * _SCB, _SCB)]
            pltpu.make_async_copy(vseg, a_h.at[i1_v.at[j]], sem1).wait()
            pltpu.make_async_copy(vseg, a_h.at[i2_v.at[j]], sem2).wait()
            return carry

        lax.fori_loop(0, n_sc, drain, 0)

    return scatter_kernel(src, dst, bvals)


_BM5 = 256


def _classifier(A, X, Wc1, Wc2, Y3):
    def body(a_ref, x_ref, w1_ref, w2_ref, y_ref, ly_ref):
        i = pl.program_id(0)
        a = a_ref[...]
        row = lax.broadcasted_iota(jnp.int32, (_BM5, N), 0) + i * _BM5
        coln = lax.broadcasted_iota(jnp.int32, (_BM5, N), 1)
        a = jnp.maximum(a, (row == coln).astype(jnp.float32))
        agg = jnp.dot(a.astype(jnp.bfloat16),
                      x_ref[...].astype(jnp.bfloat16),
                      preferred_element_type=jnp.float32)
        hy = jnp.maximum(
            jnp.dot(agg.astype(jnp.bfloat16),
                    w1_ref[...].astype(jnp.bfloat16),
                    preferred_element_type=jnp.float32),
            0.0)
        ly = jnp.dot(hy.astype(jnp.bfloat16),
                     w2_ref[...].astype(jnp.bfloat16),
                     preferred_element_type=jnp.float32)
        yb = y_ref[0, 0, :]
        lmax = jnp.max(ly, axis=1, keepdims=True)
        lse = jnp.log(jnp.sum(jnp.exp(ly - lmax), axis=1)) + lmax[:, 0]
        c10 = lax.broadcasted_iota(jnp.int32, (_BM5, CY), 1)
        lp_t = jnp.sum(jnp.where(c10 == yb[:, None], ly, 0.0), axis=1) - lse
        part = -jnp.sum(lp_t) * (1.0 / N)

        @pl.when(i == 0)
        def _():
            ly_ref[...] = jnp.zeros_like(ly_ref)

        ly_ref[...] += part[None, None]

    return pl.pallas_call(
        body,
        grid=(N // _BM5,),
        in_specs=[
            pl.BlockSpec((_BM5, N), lambda i: (i, 0)),
            pl.BlockSpec((N, DX), lambda i: (0, 0)),
            pl.BlockSpec((DX, H), lambda i: (0, 0)),
            pl.BlockSpec((H, CY), lambda i: (0, 0)),
            pl.BlockSpec((1, 1, _BM5), lambda i: (i, 0, 0)),
        ],
        out_specs=pl.BlockSpec((1, 1), lambda i: (0, 0)),
        out_shape=jax.ShapeDtypeStruct((1, 1), jnp.float32),
    )(A, X, Wc1, Wc2, Y3)


def kernel(X_one_hot_2d, A_t, Y, t_float_E, batch_src, batch_dst,
           batch_E_one_hot, W_e, b_e, W_o, b_o, Wc1, Wc2):
    src = batch_src.astype(jnp.int32)
    dst = batch_dst.astype(jnp.int32)
    W1 = W_e[:DX]
    W2 = W_e[DX:2 * DX]
    W5 = W_e[2 * DX:2 * DX + CE]
    W6 = W_e[2 * DX + CE:].reshape(1, H)
    be = b_e.reshape(1, H)
    bo = b_o.reshape(1, CE)
    gum = jax.random.gumbel(jax.random.key(42), (E, CE), jnp.float32)

    T1, T2 = _project(X_one_hot_2d, W1, W2)
    G1, G2 = _sc_gather(T1, T2, src, dst)
    bflag3, loss_e = _edge_head(G1, G2, A_t, t_float_E, W5, W6, be, W_o, bo,
                                gum, batch_E_one_hot)
    bflag = bflag3.reshape(E)
    A_flat = _sc_scatter(src, dst, bflag)
    A = A_flat.reshape(NPAD, N)
    Y3 = Y.astype(jnp.int32).reshape(N // _BM5, 1, _BM5)
    loss_y = _classifier(A, X_one_hot_2d, Wc1, Wc2, Y3)
    return loss_e[0, 0], loss_y[0, 0]


# speedup vs baseline: 12.1359x; 1.0100x over previous
import functools

import jax
import jax.numpy as jnp
from jax import lax
from jax.experimental import pallas as pl
from jax.experimental.pallas import tpu as pltpu
from jax.experimental.pallas import tpu_sc as plsc

N = 4096
E = 65536
DX = 512
H = 1024
CE = 5
CY = 10

NC = 2
NS = 16
NW = NC * NS


def _pack_bf16_pair(o):
    u = lax.bitcast_convert_type(o, jnp.uint32)
    b = (u + jnp.uint32(0x7FFF) + ((u >> 16) & jnp.uint32(1))) >> 16
    lo = b[:, :H // 2]
    hi = b[:, H // 2:]
    return lax.bitcast_convert_type(lo | (hi << 16), jnp.int32)


def _unpack_bf16_pair(gi):
    g = lax.bitcast_convert_type(gi, jnp.uint32)
    lo = lax.bitcast_convert_type(g << 16, jnp.float32)
    hi = lax.bitcast_convert_type(g & jnp.uint32(0xFFFF0000), jnp.float32)
    return jnp.concatenate([lo, hi], axis=1)


def _proj_body(x_ref, w1_ref, w2_ref, o1_ref, o2_ref):
    x = x_ref[...]
    o1_ref[...] = _pack_bf16_pair(
        jnp.dot(x, w1_ref[...], preferred_element_type=jnp.float32))
    o2_ref[...] = _pack_bf16_pair(
        jnp.dot(x, w2_ref[...], preferred_element_type=jnp.float32))


def _project(X, W1, W2):
    BM = 512
    return pl.pallas_call(
        _proj_body,
        grid=(N // BM,),
        in_specs=[
            pl.BlockSpec((BM, DX), lambda i: (i, 0)),
            pl.BlockSpec((DX, H), lambda i: (0, 0)),
            pl.BlockSpec((DX, H), lambda i: (0, 0)),
        ],
        out_specs=[
            pl.BlockSpec((BM, H // 2), lambda i: (i, 0)),
            pl.BlockSpec((BM, H // 2), lambda i: (i, 0)),
        ],
        out_shape=[jax.ShapeDtypeStruct((N, H // 2), jnp.int32)] * 2,
    )(X, W1, W2)


_CH = 32
_HW = H // 2


def _sc_gather(T1, T2, src, dst):
    mesh = plsc.VectorSubcoreMesh(core_axis_name="c", subcore_axis_name="s")
    bpw = E // NW
    nch = bpw // _CH

    @functools.partial(
        pl.kernel,
        out_type=[jax.ShapeDtypeStruct((E, _HW), jnp.int32)] * 2,
        mesh=mesh,
        scratch_types=[
            pltpu.VMEM((bpw,), jnp.int32),
            pltpu.VMEM((bpw,), jnp.int32),
            pltpu.VMEM((_CH, _HW), jnp.int32),
            pltpu.VMEM((_CH, _HW), jnp.int32),
            pltpu.VMEM((_CH, _HW), jnp.int32),
            pltpu.VMEM((_CH, _HW), jnp.int32),
            pltpu.SemaphoreType.DMA,
            pltpu.SemaphoreType.DMA,
            pltpu.SemaphoreType.DMA,
            pltpu.SemaphoreType.DMA,
        ],
    )
    def gather_kernel(t1, t2, s_h, d_h, g1, g2, s_all, d_all,
                      r1a, r2a, r1b, r2b, m1a, m2a, m1b, m2b):
        wid = lax.axis_index("s") * NC + lax.axis_index("c")
        base = wid * bpw
        pltpu.sync_copy(s_h.at[pl.ds(base, bpw)], s_all)
        pltpu.sync_copy(d_h.at[pl.ds(base, bpw)], d_all)

        def issue(c, r1, r2, m1, m2):
            pltpu.async_copy(t1.at[s_all.at[pl.ds(c * _CH, _CH)]], r1, m1)
            pltpu.async_copy(t2.at[d_all.at[pl.ds(c * _CH, _CH)]], r2, m2)

        def wait(c, r1, r2, m1, m2):
            i1 = s_all.at[pl.ds(c * _CH, _CH)]
            i2 = d_all.at[pl.ds(c * _CH, _CH)]
            pltpu.make_async_copy(t1.at[i1], r1, m1).wait()
            pltpu.make_async_copy(t2.at[i2], r2, m2).wait()

        def wout(c, r1, r2):
            off = base + c * _CH
            pltpu.sync_copy(r1, g1.at[pl.ds(off, _CH)])
            pltpu.sync_copy(r2, g2.at[pl.ds(off, _CH)])

        issue(0, r1a, r2a, m1a, m2a)

        def body(c2, carry):
            cA = 2 * c2
            cB = cA + 1
            issue(cB, r1b, r2b, m1b, m2b)
            wait(cA, r1a, r2a, m1a, m2a)
            wout(cA, r1a, r2a)

            @pl.when(cA + 2 < nch)
            def _():
                issue(cA + 2, r1a, r2a, m1a, m2a)

            wait(cB, r1b, r2b, m1b, m2b)
            wout(cB, r1b, r2b)
            return carry

        lax.fori_loop(0, nch // 2, body, 0)

    return gather_kernel(T1, T2, src, dst)


_TE = 2048


def _edge_head(G1, G2, At, tf, W5, W6, be, Wo, bo, gum, eoh):
    grid = (E // _TE,)

    def body(g1_ref, g2_ref, at_ref, t_ref, w5_ref, w6_ref, be_ref, wo_ref,
             bo_ref, gum_ref, eoh_ref, b_ref, le_ref):
        i = pl.program_id(0)
        sm = jnp.dot(at_ref[...], w5_ref[...],
                     preferred_element_type=jnp.float32)
        sm = sm + t_ref[...] * w6_ref[...]
        g = _unpack_bf16_pair(g1_ref[...]) + _unpack_bf16_pair(g2_ref[...])
        h = jnp.maximum(g + sm + be_ref[...], 0.0)
        logit = jnp.dot(h, wo_ref[...],
                        preferred_element_type=jnp.float32) + bo_ref[...]
        col = lax.broadcasted_iota(jnp.int32, (_TE, CE), 1)
        z = logit + gum_ref[...]
        zmax = jnp.max(z, axis=1, keepdims=True)
        samp = jnp.min(jnp.where(z >= zmax, col, CE), axis=1)
        b_ref[...] = (samp != 0).astype(jnp.float32)[None, None, :]
        eoh = eoh_ref[...]
        emax = jnp.max(eoh, axis=1, keepdims=True)
        te_idx = jnp.min(jnp.where(eoh >= emax, col, CE), axis=1)
        lmax = jnp.max(logit, axis=1, keepdims=True)
        lse = jnp.log(jnp.sum(jnp.exp(logit - lmax), axis=1)) + lmax[:, 0]
        lp_t = jnp.sum(jnp.where(col == te_idx[:, None], logit, 0.0),
                       axis=1) - lse
        part = -jnp.sum(lp_t) * (1.0 / E)

        @pl.when(i == 0)
        def _():
            le_ref[...] = jnp.zeros_like(le_ref)

        le_ref[...] += part[None, None]

    return pl.pallas_call(
        body,
        grid=grid,
        in_specs=[
            pl.BlockSpec((_TE, _HW), lambda i: (i, 0)),
            pl.BlockSpec((_TE, _HW), lambda i: (i, 0)),
            pl.BlockSpec((_TE, CE), lambda i: (i, 0)),
            pl.BlockSpec((_TE, 1), lambda i: (i, 0)),
            pl.BlockSpec((CE, H), lambda i: (0, 0)),
            pl.BlockSpec((1, H), lambda i: (0, 0)),
            pl.BlockSpec((1, H), lambda i: (0, 0)),
            pl.BlockSpec((H, CE), lambda i: (0, 0)),
            pl.BlockSpec((1, CE), lambda i: (0, 0)),
            pl.BlockSpec((_TE, CE), lambda i: (i, 0)),
            pl.BlockSpec((_TE, CE), lambda i: (i, 0)),
        ],
        out_specs=[
            pl.BlockSpec((1, 1, _TE), lambda i: (i, 0, 0)),
            pl.BlockSpec((1, 1), lambda i: (0, 0)),
        ],
        out_shape=[
            jax.ShapeDtypeStruct((E // _TE, 1, _TE), jnp.float32),
            jax.ShapeDtypeStruct((1, 1), jnp.float32),
        ],
    )(G1, G2, At, tf, W5, W6, be, Wo, bo, gum, eoh)


_ZCH = 16384
_SCB = 128
NPAD = N + 32


def _sc_scatter(src, dst, bvals):
    mesh = plsc.VectorSubcoreMesh(core_axis_name="c", subcore_axis_name="s")
    epw = E // NS
    n_sc = epw // _SCB
    half_words = (N // 2) * N

    @functools.partial(
        pl.kernel,
        out_type=jax.ShapeDtypeStruct((NPAD * N,), jnp.float32),
        mesh=mesh,
        scratch_types=[
            pltpu.VMEM((_ZCH,), jnp.float32),
            pltpu.VMEM((epw,), jnp.int32),
            pltpu.VMEM((epw,), jnp.int32),
            pltpu.VMEM((epw,), jnp.float32),
            pltpu.VMEM((n_sc, _SCB), jnp.int32),
            pltpu.VMEM((n_sc, _SCB), jnp.int32),
            pltpu.SemaphoreType.DMA,
            pltpu.SemaphoreType.DMA,
        ],
    )
    def scatter_kernel(s_h, d_h, b_h, a_h, z_v, s_v, d_v, v_v, i1_v, i2_v,
                       sem1, sem2):
        cid = lax.axis_index("c")
        sid = lax.axis_index("s")

        def zb(i, carry):
            z_v[pl.ds(i * 16, 16)] = jnp.zeros((16,), jnp.float32)
            return carry

        lax.fori_loop(0, _ZCH // 16, zb, 0)
        words = half_words // NS
        zbase = cid * half_words + sid * words

        def zc(i, carry):
            pltpu.sync_copy(z_v, a_h.at[pl.ds(zbase + i * _ZCH, _ZCH)])
            return carry

        lax.fori_loop(0, words // _ZCH, zc, 0)
        tzbase = N * N + cid * (16 * N) + sid * N
        pltpu.sync_copy(z_v.at[pl.ds(0, N)], a_h.at[pl.ds(tzbase, N)])

        plsc.subcore_barrier()

        ebase = sid * epw
        pltpu.sync_copy(s_h.at[pl.ds(ebase, epw)], s_v)
        pltpu.sync_copy(d_h.at[pl.ds(ebase, epw)], d_v)
        pltpu.sync_copy(b_h.at[pl.ds(ebase, epw)], v_v)

        lo = cid * (N // 2)
        hi = lo + (N // 2)
        trash = N * N + cid * (16 * N)
        lane = lax.iota(jnp.int32, 16)

        def ixrow(j, carry):
            def ix(i, c2):
                sv = s_v[pl.ds(j * _SCB + i * 16, 16)]
                dv = d_v[pl.ds(j * _SCB + i * 16, 16)]
                own1 = jnp.logical_and(dv >= lo, dv < hi)
                own2 = jnp.logical_and(sv >= lo, sv < hi)
                base_off = ebase + j * _SCB + i * 16 + lane
                tr1 = trash + jnp.bitwise_and(base_off, 16 * N - 1)
                tr2 = trash + jnp.bitwise_and(base_off + 8 * N, 16 * N - 1)
                i1_v[j, pl.ds(i * 16, 16)] = jnp.where(
                    own1, dv * N + sv, tr1)
                i2_v[j, pl.ds(i * 16, 16)] = jnp.where(
                    own2, sv * N + dv, tr2)
                return c2

            lax.fori_loop(0, _SCB // 16, ix, 0)
            return carry

        lax.fori_loop(0, n_sc, ixrow, 0)

        def sc(j, carry):
            vseg = v_v.at[pl.ds(j * _SCB, _SCB)]
            cp1 = pltpu.async_copy(vseg, a_h.at[i1_v.at[j]], sem1)
            cp2 = pltpu.async_copy(vseg, a_h.at[i2_v.at[j]], sem2)
            cp1.wait()
            cp2.wait()
            return carry

        lax.fori_loop(0, n_sc, sc, 0)

    return scatter_kernel(src, dst, bvals)


_BM5 = 256


def _classifier(A, X, Wc1, Wc2, Y3):
    def body(a_ref, x_ref, w1_ref, w2_ref, y_ref, ly_ref):
        i = pl.program_id(0)
        a = a_ref[...]
        row = lax.broadcasted_iota(jnp.int32, (_BM5, N), 0) + i * _BM5
        coln = lax.broadcasted_iota(jnp.int32, (_BM5, N), 1)
        a = jnp.maximum(a, (row == coln).astype(jnp.float32))
        agg = jnp.dot(a.astype(jnp.bfloat16),
                      x_ref[...].astype(jnp.bfloat16),
                      preferred_element_type=jnp.float32)
        hy = jnp.maximum(
            jnp.dot(agg.astype(jnp.bfloat16),
                    w1_ref[...].astype(jnp.bfloat16),
                    preferred_element_type=jnp.float32),
            0.0)
        ly = jnp.dot(hy.astype(jnp.bfloat16),
                     w2_ref[...].astype(jnp.bfloat16),
                     preferred_element_type=jnp.float32)
        yb = y_ref[0, 0, :]
        lmax = jnp.max(ly, axis=1, keepdims=True)
        lse = jnp.log(jnp.sum(jnp.exp(ly - lmax), axis=1)) + lmax[:, 0]
        c10 = lax.broadcasted_iota(jnp.int32, (_BM5, CY), 1)
        lp_t = jnp.sum(jnp.where(c10 == yb[:, None], ly, 0.0), axis=1) - lse
        part = -jnp.sum(lp_t) * (1.0 / N)

        @pl.when(i == 0)
        def _():
            ly_ref[...] = jnp.zeros_like(ly_ref)

        ly_ref[...] += part[None, None]

    return pl.pallas_call(
        body,
        grid=(N // _BM5,),
        in_specs=[
            pl.BlockSpec((_BM5, N), lambda i: (i, 0)),
            pl.BlockSpec((N, DX), lambda i: (0, 0)),
            pl.BlockSpec((DX, H), lambda i: (0, 0)),
            pl.BlockSpec((H, CY), lambda i: (0, 0)),
            pl.BlockSpec((1, 1, _BM5), lambda i: (i, 0, 0)),
        ],
        out_specs=pl.BlockSpec((1, 1), lambda i: (0, 0)),
        out_shape=jax.ShapeDtypeStruct((1, 1), jnp.float32),
    )(A, X, Wc1, Wc2, Y3)


def kernel(X_one_hot_2d, A_t, Y, t_float_E, batch_src, batch_dst,
           batch_E_one_hot, W_e, b_e, W_o, b_o, Wc1, Wc2):
    src = batch_src.astype(jnp.int32)
    dst = batch_dst.astype(jnp.int32)
    W1 = W_e[:DX]
    W2 = W_e[DX:2 * DX]
    W5 = W_e[2 * DX:2 * DX + CE]
    W6 = W_e[2 * DX + CE:].reshape(1, H)
    be = b_e.reshape(1, H)
    bo = b_o.reshape(1, CE)
    gum = jax.random.gumbel(jax.random.key(42), (E, CE), jnp.float32)

    T1, T2 = _project(X_one_hot_2d, W1, W2)
    G1, G2 = _sc_gather(T1, T2, src, dst)
    bflag3, loss_e = _edge_head(G1, G2, A_t, t_float_E, W5, W6, be, W_o, bo,
                                gum, batch_E_one_hot)
    bflag = bflag3.reshape(E)
    A_flat = _sc_scatter(src, dst, bflag)
    A = A_flat.reshape(NPAD, N)
    Y3 = Y.astype(jnp.int32).reshape(N // _BM5, 1, _BM5)
    loss_y = _classifier(A, X_one_hot_2d, Wc1, Wc2, Y3)
    return loss_e[0, 0], loss_y[0, 0]


# speedup vs baseline: 16.1651x; 1.3320x over previous
import functools

import jax
import jax.numpy as jnp
from jax import lax
from jax.experimental import pallas as pl
from jax.experimental.pallas import tpu as pltpu
from jax.experimental.pallas import tpu_sc as plsc

N = 4096
E = 65536
DX = 512
H = 1024
CE = 5
CY = 10

NC = 2
NS = 16
NW = NC * NS


def _pack_bf16_pair(o):
    u = lax.bitcast_convert_type(o, jnp.uint32)
    b = (u + jnp.uint32(0x7FFF) + ((u >> 16) & jnp.uint32(1))) >> 16
    lo = b[:, :H // 2]
    hi = b[:, H // 2:]
    return lax.bitcast_convert_type(lo | (hi << 16), jnp.int32)


def _unpack_bf16_pair(gi):
    g = lax.bitcast_convert_type(gi, jnp.uint32)
    lo = lax.bitcast_convert_type(g << 16, jnp.float32)
    hi = lax.bitcast_convert_type(g & jnp.uint32(0xFFFF0000), jnp.float32)
    return jnp.concatenate([lo, hi], axis=1)


def _proj_body(x_ref, w1_ref, w2_ref, o1_ref, o2_ref):
    x = x_ref[...]
    o1_ref[...] = _pack_bf16_pair(
        jnp.dot(x, w1_ref[...], preferred_element_type=jnp.float32))
    o2_ref[...] = _pack_bf16_pair(
        jnp.dot(x, w2_ref[...], preferred_element_type=jnp.float32))


def _project(X, W1, W2):
    BM = 512
    return pl.pallas_call(
        _proj_body,
        grid=(N // BM,),
        in_specs=[
            pl.BlockSpec((BM, DX), lambda i: (i, 0)),
            pl.BlockSpec((DX, H), lambda i: (0, 0)),
            pl.BlockSpec((DX, H), lambda i: (0, 0)),
        ],
        out_specs=[
            pl.BlockSpec((BM, H // 2), lambda i: (i, 0)),
            pl.BlockSpec((BM, H // 2), lambda i: (i, 0)),
        ],
        out_shape=[jax.ShapeDtypeStruct((N, H // 2), jnp.int32)] * 2,
    )(X, W1, W2)


_CH = 32
_HW = H // 2


def _sc_gather(T1, T2, src, dst):
    mesh = plsc.VectorSubcoreMesh(core_axis_name="c", subcore_axis_name="s")
    bpw = E // NW
    nch = bpw // _CH

    @functools.partial(
        pl.kernel,
        out_type=[jax.ShapeDtypeStruct((E, _HW), jnp.int32)] * 2,
        mesh=mesh,
        scratch_types=[
            pltpu.VMEM((bpw,), jnp.int32),
            pltpu.VMEM((bpw,), jnp.int32),
            pltpu.VMEM((_CH, _HW), jnp.int32),
            pltpu.VMEM((_CH, _HW), jnp.int32),
            pltpu.VMEM((_CH, _HW), jnp.int32),
            pltpu.VMEM((_CH, _HW), jnp.int32),
            pltpu.SemaphoreType.DMA,
            pltpu.SemaphoreType.DMA,
            pltpu.SemaphoreType.DMA,
            pltpu.SemaphoreType.DMA,
        ],
    )
    def gather_kernel(t1, t2, s_h, d_h, g1, g2, s_all, d_all,
                      r1a, r2a, r1b, r2b, m1a, m2a, m1b, m2b):
        wid = lax.axis_index("s") * NC + lax.axis_index("c")
        base = wid * bpw
        pltpu.sync_copy(s_h.at[pl.ds(base, bpw)], s_all)
        pltpu.sync_copy(d_h.at[pl.ds(base, bpw)], d_all)

        def issue(c, r1, r2, m1, m2):
            pltpu.async_copy(t1.at[s_all.at[pl.ds(c * _CH, _CH)]], r1, m1)
            pltpu.async_copy(t2.at[d_all.at[pl.ds(c * _CH, _CH)]], r2, m2)

        def wait(c, r1, r2, m1, m2):
            i1 = s_all.at[pl.ds(c * _CH, _CH)]
            i2 = d_all.at[pl.ds(c * _CH, _CH)]
            pltpu.make_async_copy(t1.at[i1], r1, m1).wait()
            pltpu.make_async_copy(t2.at[i2], r2, m2).wait()

        def wout(c, r1, r2):
            off = base + c * _CH
            pltpu.sync_copy(r1, g1.at[pl.ds(off, _CH)])
            pltpu.sync_copy(r2, g2.at[pl.ds(off, _CH)])

        issue(0, r1a, r2a, m1a, m2a)

        def body(c2, carry):
            cA = 2 * c2
            cB = cA + 1
            issue(cB, r1b, r2b, m1b, m2b)
            wait(cA, r1a, r2a, m1a, m2a)
            wout(cA, r1a, r2a)

            @pl.when(cA + 2 < nch)
            def _():
                issue(cA + 2, r1a, r2a, m1a, m2a)

            wait(cB, r1b, r2b, m1b, m2b)
            wout(cB, r1b, r2b)
            return carry

        lax.fori_loop(0, nch // 2, body, 0)

    return gather_kernel(T1, T2, src, dst)


_TE = 2048


def _edge_head(G1, G2, At, tf, W5, W6, be, Wo, bo, gum, eoh):
    grid = (E // _TE,)

    def body(g1_ref, g2_ref, at_ref, t_ref, w5_ref, w6_ref, be_ref, wo_ref,
             bo_ref, gum_ref, eoh_ref, b_ref, le_ref):
        i = pl.program_id(0)
        sm = jnp.dot(at_ref[...], w5_ref[...],
                     preferred_element_type=jnp.float32)
        sm = sm + t_ref[...] * w6_ref[...]
        g = _unpack_bf16_pair(g1_ref[...]) + _unpack_bf16_pair(g2_ref[...])
        h = jnp.maximum(g + sm + be_ref[...], 0.0)
        logit = jnp.dot(h, wo_ref[...],
                        preferred_element_type=jnp.float32) + bo_ref[...]
        col = lax.broadcasted_iota(jnp.int32, (_TE, CE), 1)
        z = logit + gum_ref[...]
        zmax = jnp.max(z, axis=1, keepdims=True)
        samp = jnp.min(jnp.where(z >= zmax, col, CE), axis=1)
        b_ref[...] = (samp != 0).astype(jnp.float32)[None, None, :]
        eoh = eoh_ref[...]
        emax = jnp.max(eoh, axis=1, keepdims=True)
        te_idx = jnp.min(jnp.where(eoh >= emax, col, CE), axis=1)
        lmax = jnp.max(logit, axis=1, keepdims=True)
        lse = jnp.log(jnp.sum(jnp.exp(logit - lmax), axis=1)) + lmax[:, 0]
        lp_t = jnp.sum(jnp.where(col == te_idx[:, None], logit, 0.0),
                       axis=1) - lse
        part = -jnp.sum(lp_t) * (1.0 / E)

        @pl.when(i == 0)
        def _():
            le_ref[...] = jnp.zeros_like(le_ref)

        le_ref[...] += part[None, None]

    return pl.pallas_call(
        body,
        grid=grid,
        in_specs=[
            pl.BlockSpec((_TE, _HW), lambda i: (i, 0)),
            pl.BlockSpec((_TE, _HW), lambda i: (i, 0)),
            pl.BlockSpec((_TE, CE), lambda i: (i, 0)),
            pl.BlockSpec((_TE, 1), lambda i: (i, 0)),
            pl.BlockSpec((CE, H), lambda i: (0, 0)),
            pl.BlockSpec((1, H), lambda i: (0, 0)),
            pl.BlockSpec((1, H), lambda i: (0, 0)),
            pl.BlockSpec((H, CE), lambda i: (0, 0)),
            pl.BlockSpec((1, CE), lambda i: (0, 0)),
            pl.BlockSpec((_TE, CE), lambda i: (i, 0)),
            pl.BlockSpec((_TE, CE), lambda i: (i, 0)),
        ],
        out_specs=[
            pl.BlockSpec((1, 1, _TE), lambda i: (i, 0, 0)),
            pl.BlockSpec((1, 1), lambda i: (0, 0)),
        ],
        out_shape=[
            jax.ShapeDtypeStruct((E // _TE, 1, _TE), jnp.float32),
            jax.ShapeDtypeStruct((1, 1), jnp.float32),
        ],
    )(G1, G2, At, tf, W5, W6, be, Wo, bo, gum, eoh)


_ZCH = 16384
_SCB = 128


def _sc_scatter(src, dst, bvals):
    mesh = plsc.VectorSubcoreMesh(core_axis_name="c", subcore_axis_name="s")
    epw = E // NS
    n_sc = epw // _SCB

    @functools.partial(
        pl.kernel,
        out_type=jax.ShapeDtypeStruct((N * N,), jnp.float32),
        mesh=mesh,
        scratch_types=[
            pltpu.VMEM((_ZCH,), jnp.float32),
            pltpu.VMEM((epw,), jnp.int32),
            pltpu.VMEM((epw,), jnp.int32),
            pltpu.VMEM((epw,), jnp.float32),
            pltpu.VMEM((n_sc, _SCB), jnp.int32),
            pltpu.VMEM((n_sc, _SCB), jnp.int32),
            pltpu.SemaphoreType.DMA,
            pltpu.SemaphoreType.DMA,
        ],
    )
    def scatter_kernel(s_h, d_h, b_h, a_h, z_v, s_v, d_v, v_v, i1_v, i2_v,
                       sem1, sem2):
        cid = lax.axis_index("c")
        sid = lax.axis_index("s")

        @pl.when(cid == 0)
        def _zero():
            def zb(i, carry):
                z_v[pl.ds(i * 16, 16)] = jnp.zeros((16,), jnp.float32)
                return carry

            lax.fori_loop(0, _ZCH // 16, zb, 0)
            words = (N * N) // NS
            zbase = sid * words

            def zc(i, carry):
                pltpu.sync_copy(z_v, a_h.at[pl.ds(zbase + i * _ZCH, _ZCH)])
                return carry

            lax.fori_loop(0, words // _ZCH, zc, 0)

        plsc.subcore_barrier()

        @pl.when(cid == 0)
        def _scatter():
            ebase = sid * epw
            pltpu.sync_copy(s_h.at[pl.ds(ebase, epw)], s_v)
            pltpu.sync_copy(d_h.at[pl.ds(ebase, epw)], d_v)
            pltpu.sync_copy(b_h.at[pl.ds(ebase, epw)], v_v)

            def ixrow(j, carry):
                def ix(i, c2):
                    sv = s_v[pl.ds(j * _SCB + i * 16, 16)]
                    dv = d_v[pl.ds(j * _SCB + i * 16, 16)]
                    i1_v[j, pl.ds(i * 16, 16)] = dv * N + sv
                    i2_v[j, pl.ds(i * 16, 16)] = sv * N + dv
                    return c2

                lax.fori_loop(0, _SCB // 16, ix, 0)
                return carry

            lax.fori_loop(0, n_sc, ixrow, 0)

            def sc(j, carry):
                vseg = v_v.at[pl.ds(j * _SCB, _SCB)]
                cp1 = pltpu.async_copy(vseg, a_h.at[i1_v.at[j]], sem1)
                cp2 = pltpu.async_copy(vseg, a_h.at[i2_v.at[j]], sem2)
                cp1.wait()
                cp2.wait()
                return carry

            lax.fori_loop(0, n_sc, sc, 0)

    return scatter_kernel(src, dst, bvals)


_BM5 = 256


def _classifier(A, X, Wc1, Wc2, Y3):
    def body(a_ref, x_ref, w1_ref, w2_ref, y_ref, ly_ref):
        i = pl.program_id(0)
        a = a_ref[...]
        row = lax.broadcasted_iota(jnp.int32, (_BM5, N), 0) + i * _BM5
        coln = lax.broadcasted_iota(jnp.int32, (_BM5, N), 1)
        a = jnp.maximum(a, (row == coln).astype(jnp.float32))
        agg = jnp.dot(a.astype(jnp.bfloat16),
                      x_ref[...].astype(jnp.bfloat16),
                      preferred_element_type=jnp.float32)
        hy = jnp.maximum(
            jnp.dot(agg.astype(jnp.bfloat16),
                    w1_ref[...].astype(jnp.bfloat16),
                    preferred_element_type=jnp.float32),
            0.0)
        ly = jnp.dot(hy.astype(jnp.bfloat16),
                     w2_ref[...].astype(jnp.bfloat16),
                     preferred_element_type=jnp.float32)
        yb = y_ref[0, 0, :]
        lmax = jnp.max(ly, axis=1, keepdims=True)
        lse = jnp.log(jnp.sum(jnp.exp(ly - lmax), axis=1)) + lmax[:, 0]
        c10 = lax.broadcasted_iota(jnp.int32, (_BM5, CY), 1)
        lp_t = jnp.sum(jnp.where(c10 == yb[:, None], ly, 0.0), axis=1) - lse
        part = -jnp.sum(lp_t) * (1.0 / N)

        @pl.when(i == 0)
        def _():
            ly_ref[...] = jnp.zeros_like(ly_ref)

        ly_ref[...] += part[None, None]

    return pl.pallas_call(
        body,
        grid=(N // _BM5,),
        in_specs=[
            pl.BlockSpec((_BM5, N), lambda i: (i, 0)),
            pl.BlockSpec((N, DX), lambda i: (0, 0)),
            pl.BlockSpec((DX, H), lambda i: (0, 0)),
            pl.BlockSpec((H, CY), lambda i: (0, 0)),
            pl.BlockSpec((1, 1, _BM5), lambda i: (i, 0, 0)),
        ],
        out_specs=pl.BlockSpec((1, 1), lambda i: (0, 0)),
        out_shape=jax.ShapeDtypeStruct((1, 1), jnp.float32),
    )(A, X, Wc1, Wc2, Y3)


def kernel(X_one_hot_2d, A_t, Y, t_float_E, batch_src, batch_dst,
           batch_E_one_hot, W_e, b_e, W_o, b_o, Wc1, Wc2):
    src = batch_src.astype(jnp.int32)
    dst = batch_dst.astype(jnp.int32)
    W1 = W_e[:DX]
    W2 = W_e[DX:2 * DX]
    W5 = W_e[2 * DX:2 * DX + CE]
    W6 = W_e[2 * DX + CE:].reshape(1, H)
    be = b_e.reshape(1, H)
    bo = b_o.reshape(1, CE)
    try:
        with jax.ensure_compile_time_eval():
            gum = jax.random.gumbel(jax.random.key(42), (E, CE),
                                    jnp.float32)
    except Exception:
        gum = jax.random.gumbel(jax.random.key(42), (E, CE), jnp.float32)

    T1, T2 = _project(X_one_hot_2d, W1, W2)
    G1, G2 = _sc_gather(T1, T2, src, dst)
    bflag3, loss_e = _edge_head(G1, G2, A_t, t_float_E, W5, W6, be, W_o, bo,
                                gum, batch_E_one_hot)
    bflag = bflag3.reshape(E)
    A = _sc_scatter(src, dst, bflag).reshape(N, N)
    Y3 = Y.astype(jnp.int32).reshape(N // _BM5, 1, _BM5)
    loss_y = _classifier(A, X_one_hot_2d, Wc1, Wc2, Y3)
    return loss_e[0, 0], loss_y[0, 0]


# speedup vs baseline: 16.6590x; 1.0306x over previous
import functools

import jax
import jax.numpy as jnp
from jax import lax
from jax.experimental import pallas as pl
from jax.experimental.pallas import tpu as pltpu
from jax.experimental.pallas import tpu_sc as plsc

N = 4096
E = 65536
DX = 512
H = 1024
CE = 5
CY = 10

NC = 2
NS = 16
NW = NC * NS


def _pack_bf16_pair(o):
    u = lax.bitcast_convert_type(o, jnp.uint32)
    b = (u + jnp.uint32(0x7FFF) + ((u >> 16) & jnp.uint32(1))) >> 16
    lo = b[:, :H // 2]
    hi = b[:, H // 2:]
    return lax.bitcast_convert_type(lo | (hi << 16), jnp.int32)


def _unpack_bf16_pair(gi):
    g = lax.bitcast_convert_type(gi, jnp.uint32)
    lo = lax.bitcast_convert_type(g << 16, jnp.float32)
    hi = lax.bitcast_convert_type(g & jnp.uint32(0xFFFF0000), jnp.float32)
    return jnp.concatenate([lo, hi], axis=1)


def _proj_body(x_ref, w1_ref, w2_ref, o1_ref, o2_ref):
    x = x_ref[...]
    o1_ref[...] = _pack_bf16_pair(
        jnp.dot(x, w1_ref[...], preferred_element_type=jnp.float32))
    o2_ref[...] = _pack_bf16_pair(
        jnp.dot(x, w2_ref[...], preferred_element_type=jnp.float32))


def _project(X, W1, W2):
    BM = 512
    return pl.pallas_call(
        _proj_body,
        grid=(N // BM,),
        in_specs=[
            pl.BlockSpec((BM, DX), lambda i: (i, 0)),
            pl.BlockSpec((DX, H), lambda i: (0, 0)),
            pl.BlockSpec((DX, H), lambda i: (0, 0)),
        ],
        out_specs=[
            pl.BlockSpec((BM, H // 2), lambda i: (i, 0)),
            pl.BlockSpec((BM, H // 2), lambda i: (i, 0)),
        ],
        out_shape=[jax.ShapeDtypeStruct((N, H // 2), jnp.int32)] * 2,
    )(X, W1, W2)


_CH = 32
_HW = H // 2


def _sc_gather(T1, T2, src, dst):
    mesh = plsc.VectorSubcoreMesh(core_axis_name="c", subcore_axis_name="s")
    bpw = E // NW
    nch = bpw // _CH

    @functools.partial(
        pl.kernel,
        out_type=[jax.ShapeDtypeStruct((E, _HW), jnp.int32)] * 2,
        mesh=mesh,
        scratch_types=[
            pltpu.VMEM((bpw,), jnp.int32),
            pltpu.VMEM((bpw,), jnp.int32),
            pltpu.VMEM((_CH, _HW), jnp.int32),
            pltpu.VMEM((_CH, _HW), jnp.int32),
            pltpu.VMEM((_CH, _HW), jnp.int32),
            pltpu.VMEM((_CH, _HW), jnp.int32),
            pltpu.SemaphoreType.DMA,
            pltpu.SemaphoreType.DMA,
            pltpu.SemaphoreType.DMA,
            pltpu.SemaphoreType.DMA,
        ],
    )
    def gather_kernel(t1, t2, s_h, d_h, g1, g2, s_all, d_all,
                      r1a, r2a, r1b, r2b, m1a, m2a, m1b, m2b):
        wid = lax.axis_index("s") * NC + lax.axis_index("c")
        base = wid * bpw
        pltpu.sync_copy(s_h.at[pl.ds(base, bpw)], s_all)
        pltpu.sync_copy(d_h.at[pl.ds(base, bpw)], d_all)

        def issue(c, r1, r2, m1, m2):
            pltpu.async_copy(t1.at[s_all.at[pl.ds(c * _CH, _CH)]], r1, m1)
            pltpu.async_copy(t2.at[d_all.at[pl.ds(c * _CH, _CH)]], r2, m2)

        def wait(c, r1, r2, m1, m2):
            i1 = s_all.at[pl.ds(c * _CH, _CH)]
            i2 = d_all.at[pl.ds(c * _CH, _CH)]
            pltpu.make_async_copy(t1.at[i1], r1, m1).wait()
            pltpu.make_async_copy(t2.at[i2], r2, m2).wait()

        def wout(c, r1, r2):
            off = base + c * _CH
            pltpu.sync_copy(r1, g1.at[pl.ds(off, _CH)])
            pltpu.sync_copy(r2, g2.at[pl.ds(off, _CH)])

        issue(0, r1a, r2a, m1a, m2a)

        def body(c2, carry):
            cA = 2 * c2
            cB = cA + 1
            issue(cB, r1b, r2b, m1b, m2b)
            wait(cA, r1a, r2a, m1a, m2a)
            wout(cA, r1a, r2a)

            @pl.when(cA + 2 < nch)
            def _():
                issue(cA + 2, r1a, r2a, m1a, m2a)

            wait(cB, r1b, r2b, m1b, m2b)
            wout(cB, r1b, r2b)
            return carry

        lax.fori_loop(0, nch // 2, body, 0)

    return gather_kernel(T1, T2, src, dst)


_TE = 2048


def _edge_head(G1, G2, At, tf, W5, W6, be, Wo, bo, gum, eoh):
    grid = (E // _TE,)

    def body(g1_ref, g2_ref, at_ref, t_ref, w5_ref, w6_ref, be_ref, wo_ref,
             bo_ref, gum_ref, eoh_ref, b_ref, le_ref):
        i = pl.program_id(0)
        sm = jnp.dot(at_ref[...], w5_ref[...],
                     preferred_element_type=jnp.float32)
        sm = sm + t_ref[...] * w6_ref[...]
        g = _unpack_bf16_pair(g1_ref[...]) + _unpack_bf16_pair(g2_ref[...])
        h = jnp.maximum(g + sm + be_ref[...], 0.0)
        logit = jnp.dot(h, wo_ref[...],
                        preferred_element_type=jnp.float32) + bo_ref[...]
        col = lax.broadcasted_iota(jnp.int32, (_TE, CE), 1)
        z = logit + gum_ref[...]
        zmax = jnp.max(z, axis=1, keepdims=True)
        samp = jnp.min(jnp.where(z >= zmax, col, CE), axis=1)
        b_ref[...] = (samp != 0).astype(jnp.float32)[None, None, :]
        eoh = eoh_ref[...]
        emax = jnp.max(eoh, axis=1, keepdims=True)
        te_idx = jnp.min(jnp.where(eoh >= emax, col, CE), axis=1)
        lmax = jnp.max(logit, axis=1, keepdims=True)
        lse = jnp.log(jnp.sum(jnp.exp(logit - lmax), axis=1)) + lmax[:, 0]
        lp_t = jnp.sum(jnp.where(col == te_idx[:, None], logit, 0.0),
                       axis=1) - lse
        part = -jnp.sum(lp_t) * (1.0 / E)

        @pl.when(i == 0)
        def _():
            le_ref[...] = jnp.zeros_like(le_ref)

        le_ref[...] += part[None, None]

    return pl.pallas_call(
        body,
        grid=grid,
        in_specs=[
            pl.BlockSpec((_TE, _HW), lambda i: (i, 0)),
            pl.BlockSpec((_TE, _HW), lambda i: (i, 0)),
            pl.BlockSpec((_TE, CE), lambda i: (i, 0)),
            pl.BlockSpec((_TE, 1), lambda i: (i, 0)),
            pl.BlockSpec((CE, H), lambda i: (0, 0)),
            pl.BlockSpec((1, H), lambda i: (0, 0)),
            pl.BlockSpec((1, H), lambda i: (0, 0)),
            pl.BlockSpec((H, CE), lambda i: (0, 0)),
            pl.BlockSpec((1, CE), lambda i: (0, 0)),
            pl.BlockSpec((_TE, CE), lambda i: (i, 0)),
            pl.BlockSpec((_TE, CE), lambda i: (i, 0)),
        ],
        out_specs=[
            pl.BlockSpec((1, 1, _TE), lambda i: (i, 0, 0)),
            pl.BlockSpec((1, 1), lambda i: (0, 0)),
        ],
        out_shape=[
            jax.ShapeDtypeStruct((E // _TE, 1, _TE), jnp.float32),
            jax.ShapeDtypeStruct((1, 1), jnp.float32),
        ],
    )(G1, G2, At, tf, W5, W6, be, Wo, bo, gum, eoh)


_ZCH = 16384
_SCB = 128


def _sc_scatter(src, dst, bvals, a_ref):
    mesh = plsc.VectorSubcoreMesh(core_axis_name="c", subcore_axis_name="s")
    epw = E // NS
    n_sc = epw // _SCB

    @functools.partial(
        pl.kernel,
        out_type=(),
        mesh=mesh,
        scratch_types=[
            pltpu.VMEM((epw,), jnp.int32),
            pltpu.VMEM((epw,), jnp.int32),
            pltpu.VMEM((epw,), jnp.float32),
            pltpu.VMEM((n_sc, _SCB), jnp.int32),
            pltpu.VMEM((n_sc, _SCB), jnp.int32),
            pltpu.SemaphoreType.DMA,
            pltpu.SemaphoreType.DMA,
        ],
    )
    def scatter_kernel(s_h, d_h, b_h, a_h, s_v, d_v, v_v, i1_v, i2_v,
                       sem1, sem2):
        cid = lax.axis_index("c")
        sid = lax.axis_index("s")

        @pl.when(cid == 0)
        def _scatter():
            ebase = sid * epw
            pltpu.sync_copy(s_h.at[pl.ds(ebase, epw)], s_v)
            pltpu.sync_copy(d_h.at[pl.ds(ebase, epw)], d_v)
            pltpu.sync_copy(b_h.at[pl.ds(ebase, epw)], v_v)

            def ixrow(j, carry):
                def ix(i, c2):
                    sv = s_v[pl.ds(j * _SCB + i * 16, 16)]
                    dv = d_v[pl.ds(j * _SCB + i * 16, 16)]
                    i1_v[j, pl.ds(i * 16, 16)] = dv * N + sv
                    i2_v[j, pl.ds(i * 16, 16)] = sv * N + dv
                    return c2

                lax.fori_loop(0, _SCB // 16, ix, 0)
                return carry

            lax.fori_loop(0, n_sc, ixrow, 0)

            def sc(j, carry):
                vseg = v_v.at[pl.ds(j * _SCB, _SCB)]
                cp1 = pltpu.async_copy(vseg, a_h.at[i1_v.at[j]], sem1)
                cp2 = pltpu.async_copy(vseg, a_h.at[i2_v.at[j]], sem2)
                cp1.wait()
                cp2.wait()
                return carry

            lax.fori_loop(0, n_sc, sc, 0)

    scatter_kernel(src, dst, bvals, a_ref)


_BM5 = 256


def _classifier(A, X, Wc1, Wc2, Y3):
    def body(a_ref, x_ref, w1_ref, w2_ref, y_ref, ly_ref):
        i = pl.program_id(0)
        a = a_ref[...]
        row = lax.broadcasted_iota(jnp.int32, (_BM5, N), 0) + i * _BM5
        coln = lax.broadcasted_iota(jnp.int32, (_BM5, N), 1)
        a = jnp.maximum(a, (row == coln).astype(jnp.float32))
        agg = jnp.dot(a.astype(jnp.bfloat16),
                      x_ref[...].astype(jnp.bfloat16),
                      preferred_element_type=jnp.float32)
        hy = jnp.maximum(
            jnp.dot(agg.astype(jnp.bfloat16),
                    w1_ref[...].astype(jnp.bfloat16),
                    preferred_element_type=jnp.float32),
            0.0)
        ly = jnp.dot(hy.astype(jnp.bfloat16),
                     w2_ref[...].astype(jnp.bfloat16),
                     preferred_element_type=jnp.float32)
        yb = y_ref[0, 0, :]
        lmax = jnp.max(ly, axis=1, keepdims=True)
        lse = jnp.log(jnp.sum(jnp.exp(ly - lmax), axis=1)) + lmax[:, 0]
        c10 = lax.broadcasted_iota(jnp.int32, (_BM5, CY), 1)
        lp_t = jnp.sum(jnp.where(c10 == yb[:, None], ly, 0.0), axis=1) - lse
        part = -jnp.sum(lp_t) * (1.0 / N)

        @pl.when(i == 0)
        def _():
            ly_ref[...] = jnp.zeros_like(ly_ref)

        ly_ref[...] += part[None, None]

    return pl.pallas_call(
        body,
        grid=(N // _BM5,),
        in_specs=[
            pl.BlockSpec((_BM5, N), lambda i: (i, 0)),
            pl.BlockSpec((N, DX), lambda i: (0, 0)),
            pl.BlockSpec((DX, H), lambda i: (0, 0)),
            pl.BlockSpec((H, CY), lambda i: (0, 0)),
            pl.BlockSpec((1, 1, _BM5), lambda i: (i, 0, 0)),
        ],
        out_specs=pl.BlockSpec((1, 1), lambda i: (0, 0)),
        out_shape=jax.ShapeDtypeStruct((1, 1), jnp.float32),
    )(A, X, Wc1, Wc2, Y3)


def kernel(X_one_hot_2d, A_t, Y, t_float_E, batch_src, batch_dst,
           batch_E_one_hot, W_e, b_e, W_o, b_o, Wc1, Wc2):
    src = batch_src.astype(jnp.int32)
    dst = batch_dst.astype(jnp.int32)
    W1 = W_e[:DX]
    W2 = W_e[DX:2 * DX]
    W5 = W_e[2 * DX:2 * DX + CE]
    W6 = W_e[2 * DX + CE:].reshape(1, H)
    be = b_e.reshape(1, H)
    bo = b_o.reshape(1, CE)
    try:
        with jax.ensure_compile_time_eval():
            gum = jax.random.gumbel(jax.random.key(42), (E, CE),
                                    jnp.float32)
    except Exception:
        gum = jax.random.gumbel(jax.random.key(42), (E, CE), jnp.float32)

    T1, T2 = _project(X_one_hot_2d, W1, W2)
    G1, G2 = _sc_gather(T1, T2, src, dst)
    bflag3, loss_e = _edge_head(G1, G2, A_t, t_float_E, W5, W6, be, W_o, bo,
                                gum, batch_E_one_hot)
    bflag = bflag3.reshape(E)
    a_ref = jax.new_ref(jnp.zeros((N * N,), jnp.float32))
    _sc_scatter(src, dst, bflag, a_ref)
    A = a_ref[...].reshape(N, N)
    Y3 = Y.astype(jnp.int32).reshape(N // _BM5, 1, _BM5)
    loss_y = _classifier(A, X_one_hot_2d, Wc1, Wc2, Y3)
    return loss_e[0, 0], loss_y[0, 0]


# speedup vs baseline: 19.3331x; 1.1605x over previous
import functools

import jax
import jax.numpy as jnp
from jax import lax
from jax.experimental import pallas as pl
from jax.experimental.pallas import tpu as pltpu
from jax.experimental.pallas import tpu_sc as plsc

N = 4096
E = 65536
DX = 512
H = 1024
CE = 5
CY = 10

NC = 2
NS = 16
NW = NC * NS


def _pack_bf16_pair(o):
    u = lax.bitcast_convert_type(o, jnp.uint32)
    b = (u + jnp.uint32(0x7FFF) + ((u >> 16) & jnp.uint32(1))) >> 16
    lo = b[:, :H // 2]
    hi = b[:, H // 2:]
    return lax.bitcast_convert_type(lo | (hi << 16), jnp.int32)


def _unpack_bf16_pair(gi):
    g = lax.bitcast_convert_type(gi, jnp.uint32)
    lo = lax.bitcast_convert_type(g << 16, jnp.float32)
    hi = lax.bitcast_convert_type(g & jnp.uint32(0xFFFF0000), jnp.float32)
    return jnp.concatenate([lo, hi], axis=1)


def _proj_body(x_ref, w1_ref, w2_ref, o1_ref, o2_ref):
    x = x_ref[...]
    o1_ref[...] = _pack_bf16_pair(
        jnp.dot(x, w1_ref[...], preferred_element_type=jnp.float32))
    o2_ref[...] = _pack_bf16_pair(
        jnp.dot(x, w2_ref[...], preferred_element_type=jnp.float32))


def _project(X, W1, W2):
    BM = 512
    return pl.pallas_call(
        _proj_body,
        grid=(N // BM,),
        in_specs=[
            pl.BlockSpec((BM, DX), lambda i: (i, 0)),
            pl.BlockSpec((DX, H), lambda i: (0, 0)),
            pl.BlockSpec((DX, H), lambda i: (0, 0)),
        ],
        out_specs=[
            pl.BlockSpec((BM, H // 2), lambda i: (i, 0)),
            pl.BlockSpec((BM, H // 2), lambda i: (i, 0)),
        ],
        out_shape=[jax.ShapeDtypeStruct((N, H // 2), jnp.int32)] * 2,
    )(X, W1, W2)


_CH = 32
_HW = H // 2


def _sc_gather(T1, T2, src, dst, eoff, ne):
    mesh = plsc.VectorSubcoreMesh(core_axis_name="c", subcore_axis_name="s")
    bpw = ne // NW
    nch = bpw // _CH

    @functools.partial(
        pl.kernel,
        out_type=[jax.ShapeDtypeStruct((ne, _HW), jnp.int32)] * 2,
        mesh=mesh,
        scratch_types=[
            pltpu.VMEM((bpw,), jnp.int32),
            pltpu.VMEM((bpw,), jnp.int32),
            pltpu.VMEM((_CH, _HW), jnp.int32),
            pltpu.VMEM((_CH, _HW), jnp.int32),
            pltpu.VMEM((_CH, _HW), jnp.int32),
            pltpu.VMEM((_CH, _HW), jnp.int32),
            pltpu.SemaphoreType.DMA,
            pltpu.SemaphoreType.DMA,
            pltpu.SemaphoreType.DMA,
            pltpu.SemaphoreType.DMA,
        ],
    )
    def gather_kernel(t1, t2, s_h, d_h, g1, g2, s_all, d_all,
                      r1a, r2a, r1b, r2b, m1a, m2a, m1b, m2b):
        wid = lax.axis_index("s") * NC + lax.axis_index("c")
        base = wid * bpw
        pltpu.sync_copy(s_h.at[pl.ds(eoff + base, bpw)], s_all)
        pltpu.sync_copy(d_h.at[pl.ds(eoff + base, bpw)], d_all)

        def issue(c, r1, r2, m1, m2):
            pltpu.async_copy(t1.at[s_all.at[pl.ds(c * _CH, _CH)]], r1, m1)
            pltpu.async_copy(t2.at[d_all.at[pl.ds(c * _CH, _CH)]], r2, m2)

        def wait(c, r1, r2, m1, m2):
            i1 = s_all.at[pl.ds(c * _CH, _CH)]
            i2 = d_all.at[pl.ds(c * _CH, _CH)]
            pltpu.make_async_copy(t1.at[i1], r1, m1).wait()
            pltpu.make_async_copy(t2.at[i2], r2, m2).wait()

        def wout(c, r1, r2):
            off = base + c * _CH
            pltpu.sync_copy(r1, g1.at[pl.ds(off, _CH)])
            pltpu.sync_copy(r2, g2.at[pl.ds(off, _CH)])

        issue(0, r1a, r2a, m1a, m2a)

        def body(c2, carry):
            cA = 2 * c2
            cB = cA + 1
            issue(cB, r1b, r2b, m1b, m2b)
            wait(cA, r1a, r2a, m1a, m2a)
            wout(cA, r1a, r2a)

            @pl.when(cA + 2 < nch)
            def _():
                issue(cA + 2, r1a, r2a, m1a, m2a)

            wait(cB, r1b, r2b, m1b, m2b)
            wout(cB, r1b, r2b)
            return carry

        lax.fori_loop(0, nch // 2, body, 0)

    return gather_kernel(T1, T2, src, dst)


_TE = 2048


def _edge_head(G1, G2, At, tf, W5, W6, be, Wo, bo, gum, eoh, boff, ne):
    grid = (ne // _TE,)

    def body(g1_ref, g2_ref, at_ref, t_ref, w5_ref, w6_ref, be_ref, wo_ref,
             bo_ref, gum_ref, eoh_ref, b_ref, le_ref):
        i = pl.program_id(0)
        sm = jnp.dot(at_ref[...], w5_ref[...],
                     preferred_element_type=jnp.float32)
        sm = sm + t_ref[...] * w6_ref[...]
        g = _unpack_bf16_pair(g1_ref[...]) + _unpack_bf16_pair(g2_ref[...])
        h = jnp.maximum(g + sm + be_ref[...], 0.0)
        logit = jnp.dot(h, wo_ref[...],
                        preferred_element_type=jnp.float32) + bo_ref[...]
        col = lax.broadcasted_iota(jnp.int32, (_TE, CE), 1)
        z = logit + gum_ref[...]
        zmax = jnp.max(z, axis=1, keepdims=True)
        samp = jnp.min(jnp.where(z >= zmax, col, CE), axis=1)
        b_ref[...] = (samp != 0).astype(jnp.float32)[None, None, :]
        eoh = eoh_ref[...]
        emax = jnp.max(eoh, axis=1, keepdims=True)
        te_idx = jnp.min(jnp.where(eoh >= emax, col, CE), axis=1)
        lmax = jnp.max(logit, axis=1, keepdims=True)
        lse = jnp.log(jnp.sum(jnp.exp(logit - lmax), axis=1)) + lmax[:, 0]
        lp_t = jnp.sum(jnp.where(col == te_idx[:, None], logit, 0.0),
                       axis=1) - lse
        part = -jnp.sum(lp_t) * (1.0 / E)

        @pl.when(i == 0)
        def _():
            le_ref[...] = jnp.zeros_like(le_ref)

        le_ref[...] += part[None, None]

    return pl.pallas_call(
        body,
        grid=grid,
        in_specs=[
            pl.BlockSpec((_TE, _HW), lambda i: (i, 0)),
            pl.BlockSpec((_TE, _HW), lambda i: (i, 0)),
            pl.BlockSpec((_TE, CE), lambda i: (i + boff, 0)),
            pl.BlockSpec((_TE, 1), lambda i: (i + boff, 0)),
            pl.BlockSpec((CE, H), lambda i: (0, 0)),
            pl.BlockSpec((1, H), lambda i: (0, 0)),
            pl.BlockSpec((1, H), lambda i: (0, 0)),
            pl.BlockSpec((H, CE), lambda i: (0, 0)),
            pl.BlockSpec((1, CE), lambda i: (0, 0)),
            pl.BlockSpec((_TE, CE), lambda i: (i + boff, 0)),
            pl.BlockSpec((_TE, CE), lambda i: (i + boff, 0)),
        ],
        out_specs=[
            pl.BlockSpec((1, 1, _TE), lambda i: (i, 0, 0)),
            pl.BlockSpec((1, 1), lambda i: (0, 0)),
        ],
        out_shape=[
            jax.ShapeDtypeStruct((ne // _TE, 1, _TE), jnp.float32),
            jax.ShapeDtypeStruct((1, 1), jnp.float32),
        ],
    )(G1, G2, At, tf, W5, W6, be, Wo, bo, gum, eoh)


_ZCH = 16384
_SCB = 128


def _sc_scatter(src, dst, bvals, a_ref, eoff, ne):
    mesh = plsc.VectorSubcoreMesh(core_axis_name="c", subcore_axis_name="s")
    epw = ne // NS
    n_sc = epw // _SCB

    @functools.partial(
        pl.kernel,
        out_type=(),
        mesh=mesh,
        scratch_types=[
            pltpu.VMEM((epw,), jnp.int32),
            pltpu.VMEM((epw,), jnp.int32),
            pltpu.VMEM((epw,), jnp.float32),
            pltpu.VMEM((n_sc, _SCB), jnp.int32),
            pltpu.VMEM((n_sc, _SCB), jnp.int32),
            pltpu.SemaphoreType.DMA,
            pltpu.SemaphoreType.DMA,
        ],
    )
    def scatter_kernel(s_h, d_h, b_h, a_h, s_v, d_v, v_v, i1_v, i2_v,
                       sem1, sem2):
        cid = lax.axis_index("c")
        sid = lax.axis_index("s")

        @pl.when(cid == 0)
        def _scatter():
            ebase = sid * epw
            pltpu.sync_copy(s_h.at[pl.ds(eoff + ebase, epw)], s_v)
            pltpu.sync_copy(d_h.at[pl.ds(eoff + ebase, epw)], d_v)
            pltpu.sync_copy(b_h.at[pl.ds(ebase, epw)], v_v)

            def ixrow(j, carry):
                def ix(i, c2):
                    sv = s_v[pl.ds(j * _SCB + i * 16, 16)]
                    dv = d_v[pl.ds(j * _SCB + i * 16, 16)]
                    i1_v[j, pl.ds(i * 16, 16)] = dv * N + sv
                    i2_v[j, pl.ds(i * 16, 16)] = sv * N + dv
                    return c2

                lax.fori_loop(0, _SCB // 16, ix, 0)
                return carry

            lax.fori_loop(0, n_sc, ixrow, 0)

            def sc(j, carry):
                vseg = v_v.at[pl.ds(j * _SCB, _SCB)]
                cp1 = pltpu.async_copy(vseg, a_h.at[i1_v.at[j]], sem1)
                cp2 = pltpu.async_copy(vseg, a_h.at[i2_v.at[j]], sem2)
                cp1.wait()
                cp2.wait()
                return carry

            lax.fori_loop(0, n_sc, sc, 0)

    scatter_kernel(src, dst, bvals, a_ref)


_BM5 = 256


def _classifier(A, X, Wc1, Wc2, Y3):
    def body(a_ref, x_ref, w1_ref, w2_ref, y_ref, ly_ref):
        i = pl.program_id(0)
        a = a_ref[...]
        row = lax.broadcasted_iota(jnp.int32, (_BM5, N), 0) + i * _BM5
        coln = lax.broadcasted_iota(jnp.int32, (_BM5, N), 1)
        a = jnp.maximum(a, (row == coln).astype(jnp.float32))
        agg = jnp.dot(a.astype(jnp.bfloat16),
                      x_ref[...].astype(jnp.bfloat16),
                      preferred_element_type=jnp.float32)
        hy = jnp.maximum(
            jnp.dot(agg.astype(jnp.bfloat16),
                    w1_ref[...].astype(jnp.bfloat16),
                    preferred_element_type=jnp.float32),
            0.0)
        ly = jnp.dot(hy.astype(jnp.bfloat16),
                     w2_ref[...].astype(jnp.bfloat16),
                     preferred_element_type=jnp.float32)
        yb = y_ref[0, 0, :]
        lmax = jnp.max(ly, axis=1, keepdims=True)
        lse = jnp.log(jnp.sum(jnp.exp(ly - lmax), axis=1)) + lmax[:, 0]
        c10 = lax.broadcasted_iota(jnp.int32, (_BM5, CY), 1)
        lp_t = jnp.sum(jnp.where(c10 == yb[:, None], ly, 0.0), axis=1) - lse
        part = -jnp.sum(lp_t) * (1.0 / N)

        @pl.when(i == 0)
        def _():
            ly_ref[...] = jnp.zeros_like(ly_ref)

        ly_ref[...] += part[None, None]

    return pl.pallas_call(
        body,
        grid=(N // _BM5,),
        in_specs=[
            pl.BlockSpec((_BM5, N), lambda i: (i, 0)),
            pl.BlockSpec((N, DX), lambda i: (0, 0)),
            pl.BlockSpec((DX, H), lambda i: (0, 0)),
            pl.BlockSpec((H, CY), lambda i: (0, 0)),
            pl.BlockSpec((1, 1, _BM5), lambda i: (i, 0, 0)),
        ],
        out_specs=pl.BlockSpec((1, 1), lambda i: (0, 0)),
        out_shape=jax.ShapeDtypeStruct((1, 1), jnp.float32),
    )(A, X, Wc1, Wc2, Y3)


def kernel(X_one_hot_2d, A_t, Y, t_float_E, batch_src, batch_dst,
           batch_E_one_hot, W_e, b_e, W_o, b_o, Wc1, Wc2):
    src = batch_src.astype(jnp.int32)
    dst = batch_dst.astype(jnp.int32)
    W1 = W_e[:DX]
    W2 = W_e[DX:2 * DX]
    W5 = W_e[2 * DX:2 * DX + CE]
    W6 = W_e[2 * DX + CE:].reshape(1, H)
    be = b_e.reshape(1, H)
    bo = b_o.reshape(1, CE)
    try:
        with jax.ensure_compile_time_eval():
            gum = jax.random.gumbel(jax.random.key(42), (E, CE),
                                    jnp.float32)
    except Exception:
        gum = jax.random.gumbel(jax.random.key(42), (E, CE), jnp.float32)

    T1, T2 = _project(X_one_hot_2d, W1, W2)
    a_ref = jax.new_ref(jnp.zeros((N * N,), jnp.float32))
    EH = E // 2
    G1a, G2a = _sc_gather(T1, T2, src, dst, 0, EH)
    G1b, G2b = _sc_gather(T1, T2, src, dst, EH, EH)
    bfa, lea = _edge_head(G1a, G2a, A_t, t_float_E, W5, W6, be, W_o, bo,
                          gum, batch_E_one_hot, 0, EH)
    _sc_scatter(src, dst, bfa.reshape(EH), a_ref, 0, EH)
    bfb, leb = _edge_head(G1b, G2b, A_t, t_float_E, W5, W6, be, W_o, bo,
                          gum, batch_E_one_hot, EH // _TE, EH)
    _sc_scatter(src, dst, bfb.reshape(EH), a_ref, EH, EH)
    A = a_ref[...].reshape(N, N)
    Y3 = Y.astype(jnp.int32).reshape(N // _BM5, 1, _BM5)
    loss_y = _classifier(A, X_one_hot_2d, Wc1, Wc2, Y3)
    return lea[0, 0] + leb[0, 0], loss_y[0, 0]
